# Initial kernel scaffold; baseline (speedup 1.0000x reference)
import jax
import jax.numpy as jnp
from jax import lax
from jax.experimental import pallas as pl
from jax.experimental.pallas import tpu as pltpu
from jax.experimental.pallas import tpu_sc as plsc

N = 10000
E = 320000
H = 128
G = 64
L = 3

_NC = 2
_NS = 16
_NW = _NC * _NS
_EW = E // _NW
_CHG = 128
_NPAIRG = 39
_GT = _NPAIRG * 2 * _CHG
_CHS = 80
_NCHS = _EW // _CHS
_NSTRIPE = N // _NS

_MESH = plsc.VectorSubcoreMesh(core_axis_name="c", subcore_axis_name="s")
_f32 = jnp.float32


def _worker_id():
    return lax.axis_index("s") * _NC + lax.axis_index("c")


def _gather_sc(D):
    def body(tab, idx_hbm, out, idx_v, rows, sem_g, sem_o):
        w = _worker_id()
        base = w * _EW
        pltpu.sync_copy(idx_hbm.at[pl.ds(base, _EW)], idx_v)

        def drain_out(n):
            for _ in range(n):
                pltpu.make_async_copy(rows.at[0], out.at[pl.ds(0, _CHG)], sem_o).wait()

        def pair(j, carry):
            k0 = 2 * j

            @pl.when(j > 0)
            def _():
                drain_out(2)

            h0 = pltpu.async_copy(
                tab.at[idx_v.at[pl.ds(k0 * _CHG, _CHG)]], rows.at[0], sem_g)
            h1 = pltpu.async_copy(
                tab.at[idx_v.at[pl.ds((k0 + 1) * _CHG, _CHG)]], rows.at[1], sem_g)
            h0.wait()
            pltpu.async_copy(rows.at[0], out.at[pl.ds(base + k0 * _CHG, _CHG)], sem_o)
            h1.wait()
            pltpu.async_copy(
                rows.at[1], out.at[pl.ds(base + (k0 + 1) * _CHG, _CHG)], sem_o)
            return carry

        lax.fori_loop(0, _NPAIRG, pair, 0)
        drain_out(2)
        tail = _EW - _GT
        pltpu.async_copy(
            tab.at[idx_v.at[pl.ds(_GT, tail)]], rows.at[0, pl.ds(0, tail)], sem_g
        ).wait()
        pltpu.sync_copy(rows.at[0, pl.ds(0, tail)], out.at[pl.ds(base + _GT, tail)])

    return pl.kernel(
        body,
        out_type=jax.ShapeDtypeStruct((E, D), _f32),
        mesh=_MESH,
        scratch_types=[
            pltpu.VMEM((_EW,), jnp.int32),
            pltpu.VMEM((2, _CHG, D), _f32),
            pltpu.SemaphoreType.DMA,
            pltpu.SemaphoreType.DMA,
        ],
    )


_gather128 = _gather_sc(128)
_gather256 = _gather_sc(256)


def _scatter_body(src, idx_hbm, out, idx_v, rows, acc, zbuf, sem_l):
    w = _worker_id()
    c = lax.axis_index("c")
    s = lax.axis_index("s")

    z = jnp.zeros((16,), _f32)

    def zb(t, carry):
        zbuf[t // 8, pl.ds((t % 8) * 16, 16)] = z
        return carry

    lax.fori_loop(0, 1000, zb, 0)
    for r in range(5):
        pltpu.sync_copy(zbuf, acc.at[pl.ds(s * _NSTRIPE + r * 125, 125)])
    plsc.subcore_barrier()

    pltpu.sync_copy(idx_hbm.at[w], idx_v)

    def load(k, b):
        return pltpu.async_copy(src.at[pl.ds(w * _EW + k * _CHS, _CHS)], rows.at[b], sem_l)

    def drain_load():
        pltpu.make_async_copy(src.at[pl.ds(0, _CHS)], rows.at[0], sem_l).wait()

    load(0, 0)

    def pair(j, carry):
        k0 = 2 * j
        drain_load()
        load(k0 + 1, 1)
        pltpu.sync_copy(rows.at[0], acc.at[idx_v.at[k0]], add=True)
        drain_load()
        load(k0 + 2, 0)
        pltpu.sync_copy(rows.at[1], acc.at[idx_v.at[k0 + 1]], add=True)
        return carry

    lax.fori_loop(0, (_NCHS - 1) // 2, pair, 0)
    drain_load()
    pltpu.sync_copy(rows.at[0], acc.at[idx_v.at[_NCHS - 1]], add=True)
    plsc.subcore_barrier()
    pltpu.sync_copy(acc.at[pl.ds(s * _NSTRIPE, _NSTRIPE)],
                    out.at[c, pl.ds(s * _NSTRIPE, _NSTRIPE)])


_scatter = pl.kernel(
    _scatter_body,
    out_type=jax.ShapeDtypeStruct((_NC, N, H), _f32),
    mesh=_MESH,
    scratch_types=[
        pltpu.VMEM((_NCHS, _CHS), jnp.int32),
        pltpu.VMEM((2, _CHS, H), _f32),
        pltpu.VMEM_SHARED((N, H), _f32),
        pltpu.VMEM((125, H), _f32),
        pltpu.SemaphoreType.DMA,
    ],
)


def _deg_body(idx_hbm, out, idx_v, ones_v, acc, zbuf):
    w = _worker_id()
    c = lax.axis_index("c")
    s = lax.axis_index("s")

    z = jnp.zeros((16,), _f32)
    o = jnp.ones((16,), _f32)

    def fill(t, carry):
        zbuf[t] = z
        return carry

    lax.fori_loop(0, 125, fill, 0)

    def fill2(t, carry):
        ones_v[t] = o
        return carry

    lax.fori_loop(0, _CHS, fill2, 0)
    for r in range(5):
        pltpu.sync_copy(zbuf, acc.at[pl.ds(s * _NSTRIPE + r * 125, 125)])
    plsc.subcore_barrier()

    pltpu.sync_copy(idx_hbm.at[w], idx_v)

    def chunk(k, carry):
        pltpu.sync_copy(ones_v, acc.at[idx_v.at[k]], add=True)
        return carry

    lax.fori_loop(0, _NCHS, chunk, 0)
    plsc.subcore_barrier()
    pltpu.sync_copy(acc.at[pl.ds(s * _NSTRIPE, _NSTRIPE)],
                    out.at[c, pl.ds(s * _NSTRIPE, _NSTRIPE)])


_deg = pl.kernel(
    _deg_body,
    out_type=jax.ShapeDtypeStruct((_NC, N, 16), _f32),
    mesh=_MESH,
    scratch_types=[
        pltpu.VMEM((_NCHS, _CHS), jnp.int32),
        pltpu.VMEM((_CHS, 16), _f32),
        pltpu.VMEM_SHARED((N, 16), _f32),
        pltpu.VMEM((125, 16), _f32),
    ],
)


def _pool_body(s2, deg16, batch_hbm, out_s, out_d, bidx_v, rows_s, rows_d,
               acc_s, acc_d, zb_s, zb_d):
    w = _worker_id()
    c = lax.axis_index("c")
    s = lax.axis_index("s")

    z = jnp.zeros((16,), _f32)

    def zfill(t, carry):
        zb_s[t // 8, pl.ds((t % 8) * 16, 16)] = z
        return carry

    lax.fori_loop(0, 32, zfill, 0)

    def zfill2(t, carry):
        zb_d[t] = z
        return carry

    lax.fori_loop(0, 4, zfill2, 0)
    pltpu.sync_copy(zb_s, acc_s.at[pl.ds(s * 4, 4)])
    pltpu.sync_copy(zb_d, acc_d.at[pl.ds(s * 4, 4)])
    plsc.subcore_barrier()

    pltpu.sync_copy(batch_hbm, bidx_v)
    for jj in range(4):
        k = w + _NW * jj

        @pl.when(k < 125)
        def _():
            pltpu.sync_copy(s2.at[pl.ds(k * _CHS, _CHS)], rows_s)
            pltpu.sync_copy(deg16.at[pl.ds(k * _CHS, _CHS)], rows_d)
            pltpu.sync_copy(rows_s, acc_s.at[bidx_v.at[k]], add=True)
            pltpu.sync_copy(rows_d, acc_d.at[bidx_v.at[k]], add=True)

    plsc.subcore_barrier()
    pltpu.sync_copy(acc_s.at[pl.ds(s * 4, 4)], out_s.at[c, pl.ds(s * 4, 4)])
    pltpu.sync_copy(acc_d.at[pl.ds(s * 4, 4)], out_d.at[c, pl.ds(s * 4, 4)])


_pool = pl.kernel(
    _pool_body,
    out_type=[jax.ShapeDtypeStruct((_NC, G, H), _f32),
              jax.ShapeDtypeStruct((_NC, G, 16), _f32)],
    mesh=_MESH,
    scratch_types=[
        pltpu.VMEM((125, _CHS), jnp.int32),
        pltpu.VMEM((_CHS, H), _f32),
        pltpu.VMEM((_CHS, 16), _f32),
        pltpu.VMEM_SHARED((G, H), _f32),
        pltpu.VMEM_SHARED((G, 16), _f32),
        pltpu.VMEM((4, H), _f32),
        pltpu.VMEM((4, 16), _f32),
    ],
)


_BE = 2000
_BN = 2000


def _mm(a, b):
    return jnp.dot(a, b, preferred_element_type=_f32)


def _proj_init_body(x_ref, m_ref, degp_ref, trn_ref, tc_ref, deg_ref):
    t = _mm(x_ref[...], m_ref[...])
    trn_ref[...] = t[:, :256]
    tc_ref[...] = t[:, 256:]
    deg_ref[...] = degp_ref[0] + degp_ref[1]


def _proj_init(x, mcat, degp):
    return pl.pallas_call(
        _proj_init_body,
        grid=(N // _BN,),
        in_specs=[
            pl.BlockSpec((_BN, H), lambda i: (i, 0)),
            pl.BlockSpec((H, 384), lambda i: (0, 0)),
            pl.BlockSpec((_NC, _BN, 16), lambda i: (0, i, 0)),
        ],
        out_specs=[
            pl.BlockSpec((_BN, 256), lambda i: (i, 0)),
            pl.BlockSpec((_BN, H), lambda i: (i, 0)),
            pl.BlockSpec((_BN, 16), lambda i: (i, 0)),
        ],
        out_shape=[
            jax.ShapeDtypeStruct((N, 256), _f32),
            jax.ShapeDtypeStruct((N, H), _f32),
            jax.ShapeDtypeStruct((N, 16), _f32),
        ],
    )(x, mcat, degp)


def _edge_a_body(grn_ref, gc_ref, he_ref, m1_ref, m2_ref, v1_ref, v2_ref,
                 u1_ref, h2_ref):
    grn = grn_ref[...]
    u1 = jnp.maximum(
        grn[:, :H] + gc_ref[...] + _mm(he_ref[...], m1_ref[...]) + v1_ref[...], 0.0)
    u1_ref[...] = u1
    h2_ref[...] = jnp.maximum(
        grn[:, H:] + _mm(u1, m2_ref[...]) + v2_ref[...], 0.0)


def _edge_a(grn, gc, he, m1, m2, v1, v2):
    dh = he.shape[1]
    return pl.pallas_call(
        _edge_a_body,
        grid=(E // _BE,),
        in_specs=[
            pl.BlockSpec((_BE, 256), lambda i: (i, 0)),
            pl.BlockSpec((_BE, H), lambda i: (i, 0)),
            pl.BlockSpec((_BE, dh), lambda i: (i, 0)),
            pl.BlockSpec((dh, H), lambda i: (0, 0)),
            pl.BlockSpec((H, H), lambda i: (0, 0)),
            pl.BlockSpec((1, H), lambda i: (0, 0)),
            pl.BlockSpec((1, H), lambda i: (0, 0)),
        ],
        out_specs=[
            pl.BlockSpec((_BE, H), lambda i: (i, 0)),
            pl.BlockSpec((_BE, H), lambda i: (i, 0)),
        ],
        out_shape=[
            jax.ShapeDtypeStruct((E, H), _f32),
            jax.ShapeDtypeStruct((E, H), _f32),
        ],
    )(grn, gc, he, m1, m2, v1, v2)


def _edge_b_body(gn_ref, u1_ref, m_ref, v_ref, o_ref):
    o_ref[...] = jnp.maximum(
        gn_ref[...] + _mm(u1_ref[...], m_ref[...]) + v_ref[...], 0.0)


def _edge_b(gn, u1, m, v):
    return pl.pallas_call(
        _edge_b_body,
        grid=(E // _BE,),
        in_specs=[
            pl.BlockSpec((_BE, H), lambda i: (i, 0)),
            pl.BlockSpec((_BE, H), lambda i: (i, 0)),
            pl.BlockSpec((H, H), lambda i: (0, 0)),
            pl.BlockSpec((1, H), lambda i: (0, 0)),
        ],
        out_specs=pl.BlockSpec((_BE, H), lambda i: (i, 0)),
        out_shape=jax.ShapeDtypeStruct((E, H), _f32),
    )(gn, u1, m, v)


def _edge_c_body(g1_ref, g2_ref, u1_ref, m_ref, v_ref, o_ref):
    o_ref[...] = jnp.maximum(
        g1_ref[...] + g2_ref[...] + _mm(u1_ref[...], m_ref[...]) + v_ref[...], 0.0)


def _edge_c(g1, g2, u1, m, v):
    return pl.pallas_call(
        _edge_c_body,
        grid=(E // _BE,),
        in_specs=[
            pl.BlockSpec((_BE, H), lambda i: (i, 0)),
            pl.BlockSpec((_BE, H), lambda i: (i, 0)),
            pl.BlockSpec((_BE, H), lambda i: (i, 0)),
            pl.BlockSpec((H, H), lambda i: (0, 0)),
            pl.BlockSpec((1, H), lambda i: (0, 0)),
        ],
        out_specs=pl.BlockSpec((_BE, H), lambda i: (i, 0)),
        out_shape=jax.ShapeDtypeStruct((E, H), _f32),
    )(g1, g2, u1, m, v)


def _node1_body(p_ref, deg_ref, m_ref, v_ref, o_ref):
    s = p_ref[0] + p_ref[1]
    o_ref[...] = _mm(s, m_ref[...]) + deg_ref[:, 0:1] * v_ref[...]


def _node1(p, deg16, m, v):
    return pl.pallas_call(
        _node1_body,
        grid=(N // _BN,),
        in_specs=[
            pl.BlockSpec((_NC, _BN, H), lambda i: (0, i, 0)),
            pl.BlockSpec((_BN, 16), lambda i: (i, 0)),
            pl.BlockSpec((H, H), lambda i: (0, 0)),
            pl.BlockSpec((1, H), lambda i: (0, 0)),
        ],
        out_specs=pl.BlockSpec((_BN, H), lambda i: (i, 0)),
        out_shape=jax.ShapeDtypeStruct((N, H), _f32),
    )(p, deg16, m, v)


def _node2_body(p_ref, deg_ref, m_ref, v_ref, trn_ref, tc_ref, tr4_ref, tc4_ref):
    s = p_ref[0] + p_ref[1]
    t = _mm(s, m_ref[...]) + deg_ref[:, 0:1] * v_ref[...]
    trn_ref[...] = t[:, 0:256]
    tc_ref[...] = t[:, 256:384]
    tr4_ref[...] = t[:, 384:512]
    tc4_ref[...] = t[:, 512:640]


def _node2(p, deg16, mcat, vcat):
    return pl.pallas_call(
        _node2_body,
        grid=(N // _BN,),
        in_specs=[
            pl.BlockSpec((_NC, _BN, H), lambda i: (0, i, 0)),
            pl.BlockSpec((_BN, 16), lambda i: (i, 0)),
            pl.BlockSpec((H, 640), lambda i: (0, 0)),
            pl.BlockSpec((1, 640), lambda i: (0, 0)),
        ],
        out_specs=[
            pl.BlockSpec((_BN, 256), lambda i: (i, 0)),
            pl.BlockSpec((_BN, H), lambda i: (i, 0)),
            pl.BlockSpec((_BN, H), lambda i: (i, 0)),
            pl.BlockSpec((_BN, H), lambda i: (i, 0)),
        ],
        out_shape=[
            jax.ShapeDtypeStruct((N, 256), _f32),
            jax.ShapeDtypeStruct((N, H), _f32),
            jax.ShapeDtypeStruct((N, H), _f32),
            jax.ShapeDtypeStruct((N, H), _f32),
        ],
    )(p, deg16, mcat, vcat)


def _add2_body(p_ref, o_ref):
    o_ref[...] = p_ref[0] + p_ref[1]


def _add2(p):
    return pl.pallas_call(
        _add2_body,
        grid=(N // _BN,),
        in_specs=[pl.BlockSpec((_NC, _BN, H), lambda i: (0, i, 0))],
        out_specs=pl.BlockSpec((_BN, H), lambda i: (i, 0)),
        out_shape=jax.ShapeDtypeStruct((N, H), _f32),
    )(p)


def _final_body(sp_ref, sdp_ref, b2_ref, bb2_ref, w1t_ref, b1_ref, w2t_ref,
                b2r_ref, o_ref):
    s = sp_ref[0] + sp_ref[1]
    sd = sdp_ref[0][:, 0:1] + sdp_ref[1][:, 0:1]
    g = _mm(s, b2_ref[...]) + sd * bb2_ref[...]
    a = jnp.maximum(_mm(g, w1t_ref[...]) + b1_ref[...], 0.0)
    o_ref[...] = _mm(a, w2t_ref[...]) + b2r_ref[...]


def _final(sp, sdp, b2, bb2, w1t, b1, w2t, b2r):
    return pl.pallas_call(
        _final_body,
        out_shape=jax.ShapeDtypeStruct((G, 1), _f32),
    )(sp, sdp, b2, bb2, w1t, b1, w2t, b2r)


def _hp(a, b):
    return jnp.dot(a, b, precision=lax.Precision.HIGHEST)


def kernel(x, edge_attr, edge_index, batch, edge_enc_W, edge_enc_b,
           etn_edge_W1, etn_edge_b1, etn_edge_W2, etn_edge_b2,
           etn_node_W1, etn_node_b1, etn_node_W2, etn_node_b2,
           nte_node_W1, nte_node_b1, nte_node_W2, nte_node_b2,
           nte_edge_W1, nte_edge_b1, nte_edge_W2, nte_edge_b2,
           reg_W1, reg_b1, reg_W2, reg_b2):
    row = edge_index[0]
    col = edge_index[1]
    col3 = col.reshape(_NW, _NCHS, _CHS)
    batch2 = batch.reshape(125, _CHS)

    sp3e = [jnp.split(etn_edge_W1[l].T, 3, axis=0) for l in range(L)]
    sp2n = [jnp.split(etn_node_W1[l].T, 2, axis=0) for l in range(L)]
    sp2n2 = [jnp.split(nte_node_W1[l].T, 2, axis=0) for l in range(L)]
    sp3e2 = [jnp.split(nte_edge_W1[l].T, 3, axis=0) for l in range(L)]

    degp = _deg(col3)

    mcat0 = jnp.concatenate([sp3e[0][0], sp2n[0][0], sp3e[0][1]], axis=1)
    t_rn, t_c, deg16 = _proj_init(x, mcat0, degp)

    he = edge_attr
    a_fold = edge_enc_W.T
    c_fold = edge_enc_b

    out = None
    for l in range(L):
        rr_e, rc_e, re_e = sp3e[l]
        rx_n, re_n = sp2n[l]
        rx_n2, re_n2 = sp2n2[l]
        w2e_t = etn_edge_W2[l].T

        m1 = _hp(a_fold, re_e)
        v1 = (_hp(c_fold.reshape(1, -1), re_e) + etn_edge_b1[l]).reshape(1, H)
        m2 = _hp(w2e_t, re_n)
        v2 = (_hp(etn_edge_b2[l].reshape(1, -1), re_n) + etn_node_b1[l]).reshape(1, H)
        m3 = _hp(w2e_t, re_n2)
        v3 = (_hp(etn_edge_b2[l].reshape(1, -1), re_n2) + nte_node_b1[l]).reshape(1, H)

        grn = _gather256(t_rn, row)
        gc = _gather128(t_c, col)
        u1, h2 = _edge_a(grn, gc, he, m1, m2, v1, v2)
        s1p = _scatter(h2, col3)

        m5 = _hp(etn_node_W2[l].T, rx_n2)
        v5 = _hp(etn_node_b2[l].reshape(1, -1), rx_n2).reshape(1, H)
        t_n3 = _node1(s1p, deg16, m5, v5)
        gn3 = _gather128(t_n3, row)
        h3 = _edge_b(gn3, u1, m3, v3)
        s2p = _scatter(h3, col3)

        if l < L - 1:
            rr_e2, rc_e2, _ = sp3e2[l]
            b_mat = nte_node_W2[l].T
            bb = nte_node_b2[l].reshape(1, -1)
            nrr_e, nrc_e, _ = sp3e[l + 1]
            nrx_n, _ = sp2n[l + 1]
            mcat = jnp.concatenate(
                [_hp(b_mat, nrr_e), _hp(b_mat, nrx_n), _hp(b_mat, nrc_e),
                 _hp(b_mat, rr_e2), _hp(b_mat, rc_e2)], axis=1)
            vcat = jnp.concatenate(
                [_hp(bb, nrr_e), _hp(bb, nrx_n), _hp(bb, nrc_e),
                 _hp(bb, rr_e2), _hp(bb, rc_e2)], axis=1)
            t_rn, t_c, t_r4, t_c4 = _node2(s2p, deg16, mcat, vcat)

            m4 = _hp(w2e_t, sp3e2[l][2])
            v4 = (_hp(etn_edge_b2[l].reshape(1, -1), sp3e2[l][2])
                  + nte_edge_b1[l]).reshape(1, H)
            gr4 = _gather128(t_r4, row)
            gc4 = _gather128(t_c4, col)
            he = _edge_c(gr4, gc4, u1, m4, v4)
            a_fold = nte_edge_W2[l].T
            c_fold = nte_edge_b2[l]
        else:
            s2 = _add2(s2p)
            sp, sdp = _pool(s2, deg16, batch2)
            out = _final(sp, sdp, nte_node_W2[l].T, nte_node_b2[l].reshape(1, H),
                         reg_W1.T, reg_b1.reshape(1, H), reg_W2.T,
                         reg_b2.reshape(1, 1))
    return out.reshape(G)


# speedup vs baseline: 2.0006x; 2.0006x over previous
import jax
import jax.numpy as jnp
from jax import lax
from jax.experimental import pallas as pl
from jax.experimental.pallas import tpu as pltpu
from jax.experimental.pallas import tpu_sc as plsc

N = 10000
E = 320000
H = 128
G = 64
L = 3

_NC = 2
_NS = 16
_NW = _NC * _NS
_EW = E // _NW
_CHG = 128
_NPAIRG = 39
_GT = _NPAIRG * 2 * _CHG
_CHS = 80
_NCHS = _EW // _CHS
_NCHN = N // _CHS

_MESH = plsc.VectorSubcoreMesh(core_axis_name="c", subcore_axis_name="s")
_f32 = jnp.float32


def _worker_id():
    return lax.axis_index("s") * _NC + lax.axis_index("c")


def _gather_sc(D):
    def body(tab, idx_hbm, out, idx_v, rows, sem_g, sem_o):
        w = _worker_id()
        base = w * _EW
        pltpu.sync_copy(idx_hbm.at[pl.ds(base, _EW)], idx_v)

        def drain_out(n):
            for _ in range(n):
                pltpu.make_async_copy(rows.at[0], out.at[pl.ds(0, _CHG)], sem_o).wait()

        def pair(j, carry):
            k0 = 2 * j

            @pl.when(j > 0)
            def _():
                drain_out(2)

            h0 = pltpu.async_copy(
                tab.at[idx_v.at[pl.ds(k0 * _CHG, _CHG)]], rows.at[0], sem_g)
            h1 = pltpu.async_copy(
                tab.at[idx_v.at[pl.ds((k0 + 1) * _CHG, _CHG)]], rows.at[1], sem_g)
            h0.wait()
            pltpu.async_copy(rows.at[0], out.at[pl.ds(base + k0 * _CHG, _CHG)], sem_o)
            h1.wait()
            pltpu.async_copy(
                rows.at[1], out.at[pl.ds(base + (k0 + 1) * _CHG, _CHG)], sem_o)
            return carry

        lax.fori_loop(0, _NPAIRG, pair, 0)
        drain_out(2)
        tail = _EW - _GT
        pltpu.async_copy(
            tab.at[idx_v.at[pl.ds(_GT, tail)]], rows.at[0, pl.ds(0, tail)], sem_g
        ).wait()
        pltpu.sync_copy(rows.at[0, pl.ds(0, tail)], out.at[pl.ds(base + _GT, tail)])

    return pl.kernel(
        body,
        out_type=jax.ShapeDtypeStruct((E, D), _f32),
        mesh=_MESH,
        scratch_types=[
            pltpu.VMEM((_EW,), jnp.int32),
            pltpu.VMEM((2, _CHG, D), _f32),
            pltpu.SemaphoreType.DMA,
            pltpu.SemaphoreType.DMA,
        ],
    )


_gather128 = _gather_sc(128)
_gather256 = _gather_sc(256)


def _scatter_body(src, idx_hbm, out, idx_v, rows, acc, zbuf, sem_l):
    w = _worker_id()
    c = lax.axis_index("c")
    s = lax.axis_index("s")

    z = jnp.zeros((16,), _f32)

    def zb(t, carry):
        zbuf[t // 8, pl.ds((t % 8) * 16, 16)] = z
        return carry

    lax.fori_loop(0, _CHS * H // 16, zb, 0)
    for r in range(8):
        k = s + _NS * r

        @pl.when(k < _NCHN)
        def _():
            pltpu.sync_copy(zbuf, acc.at[pl.ds(k * _CHS, _CHS)])

    plsc.subcore_barrier()

    pltpu.sync_copy(idx_hbm.at[w], idx_v)

    def load(k, b):
        return pltpu.async_copy(src.at[pl.ds(w * _EW + k * _CHS, _CHS)], rows.at[b], sem_l)

    def drain_load():
        pltpu.make_async_copy(src.at[pl.ds(0, _CHS)], rows.at[0], sem_l).wait()

    load(0, 0)

    def pair(j, carry):
        k0 = 2 * j
        drain_load()
        load(k0 + 1, 1)
        pltpu.sync_copy(rows.at[0], acc.at[idx_v.at[k0]], add=True)
        drain_load()
        load(k0 + 2, 0)
        pltpu.sync_copy(rows.at[1], acc.at[idx_v.at[k0 + 1]], add=True)
        return carry

    lax.fori_loop(0, (_NCHS - 1) // 2, pair, 0)
    drain_load()
    pltpu.sync_copy(rows.at[0], acc.at[idx_v.at[_NCHS - 1]], add=True)
    plsc.subcore_barrier()
    for r in range(8):
        k = s + _NS * r

        @pl.when(k < _NCHN)
        def _():
            pltpu.sync_copy(acc.at[pl.ds(k * _CHS, _CHS)],
                            out.at[c, pl.ds(k * _CHS, _CHS)])


_scatter = pl.kernel(
    _scatter_body,
    out_type=jax.ShapeDtypeStruct((_NC, N, H), _f32),
    mesh=_MESH,
    scratch_types=[
        pltpu.VMEM((_NCHS, _CHS), jnp.int32),
        pltpu.VMEM((2, _CHS, H), _f32),
        pltpu.VMEM_SHARED((N, H), _f32),
        pltpu.VMEM((_CHS, H), _f32),
        pltpu.SemaphoreType.DMA,
    ],
)


def _deg_body(idx_hbm, out, idx_v, ones_v, acc, zbuf):
    w = _worker_id()
    c = lax.axis_index("c")
    s = lax.axis_index("s")

    z = jnp.zeros((16,), _f32)
    o = jnp.ones((16,), _f32)

    def fill(t, carry):
        zbuf[t // 8, pl.ds((t % 8) * 16, 16)] = z
        return carry

    lax.fori_loop(0, _CHS * H // 16, fill, 0)

    def fill2(t, carry):
        ones_v[t // 8, pl.ds((t % 8) * 16, 16)] = o
        return carry

    lax.fori_loop(0, _CHS * H // 16, fill2, 0)
    for r in range(8):
        k = s + _NS * r

        @pl.when(k < _NCHN)
        def _():
            pltpu.sync_copy(zbuf, acc.at[pl.ds(k * _CHS, _CHS)])

    plsc.subcore_barrier()

    pltpu.sync_copy(idx_hbm.at[w], idx_v)

    def chunk(k, carry):
        pltpu.sync_copy(ones_v, acc.at[idx_v.at[k]], add=True)
        return carry

    lax.fori_loop(0, _NCHS, chunk, 0)
    plsc.subcore_barrier()
    for r in range(8):
        k = s + _NS * r

        @pl.when(k < _NCHN)
        def _():
            pltpu.sync_copy(acc.at[pl.ds(k * _CHS, _CHS)],
                            out.at[c, pl.ds(k * _CHS, _CHS)])


_deg = pl.kernel(
    _deg_body,
    out_type=jax.ShapeDtypeStruct((_NC, N, H), _f32),
    mesh=_MESH,
    scratch_types=[
        pltpu.VMEM((_NCHS, _CHS), jnp.int32),
        pltpu.VMEM((_CHS, H), _f32),
        pltpu.VMEM_SHARED((N, H), _f32),
        pltpu.VMEM((_CHS, H), _f32),
    ],
)


def _pool_body(s2, deg16, batch_hbm, out_s, out_d, bidx_v, rows_s, rows_d,
               acc_s, acc_d, zb_s):
    w = _worker_id()
    c = lax.axis_index("c")
    s = lax.axis_index("s")

    z = jnp.zeros((16,), _f32)

    def zfill(t, carry):
        zb_s[t // 8, pl.ds((t % 8) * 16, 16)] = z
        return carry

    lax.fori_loop(0, 64, zfill, 0)

    @pl.when(s < 8)
    def _():
        pltpu.sync_copy(zb_s, acc_s.at[pl.ds(s * 8, 8)])
        pltpu.sync_copy(zb_s, acc_d.at[pl.ds(s * 8, 8)])

    plsc.subcore_barrier()

    pltpu.sync_copy(batch_hbm, bidx_v)
    for jj in range(4):
        k = w + _NW * jj

        @pl.when(k < 125)
        def _():
            pltpu.sync_copy(s2.at[pl.ds(k * _CHS, _CHS)], rows_s)
            pltpu.sync_copy(deg16.at[pl.ds(k * _CHS, _CHS)], rows_d)
            pltpu.sync_copy(rows_s, acc_s.at[bidx_v.at[k]], add=True)
            pltpu.sync_copy(rows_d, acc_d.at[bidx_v.at[k]], add=True)

    plsc.subcore_barrier()

    @pl.when(s < 8)
    def _():
        pltpu.sync_copy(acc_s.at[pl.ds(s * 8, 8)], out_s.at[c, pl.ds(s * 8, 8)])
        pltpu.sync_copy(acc_d.at[pl.ds(s * 8, 8)], out_d.at[c, pl.ds(s * 8, 8)])


_pool = pl.kernel(
    _pool_body,
    out_type=[jax.ShapeDtypeStruct((_NC, G, H), _f32),
              jax.ShapeDtypeStruct((_NC, G, H), _f32)],
    mesh=_MESH,
    scratch_types=[
        pltpu.VMEM((125, _CHS), jnp.int32),
        pltpu.VMEM((_CHS, H), _f32),
        pltpu.VMEM((_CHS, H), _f32),
        pltpu.VMEM_SHARED((G, H), _f32),
        pltpu.VMEM_SHARED((G, H), _f32),
        pltpu.VMEM((8, H), _f32),
    ],
)


_BE = 2000
_BN = 2000


def _mm(a, b):
    return jnp.dot(a, b, preferred_element_type=_f32,
                   precision=lax.Precision.HIGHEST)


def _proj_init_body(x_ref, m_ref, degp_ref, trn_ref, tc_ref, deg_ref):
    t = _mm(x_ref[...], m_ref[...])
    trn_ref[...] = t[:, :256]
    tc_ref[...] = t[:, 256:]
    deg_ref[...] = degp_ref[0] + degp_ref[1]


def _proj_init(x, mcat, degp):
    return pl.pallas_call(
        _proj_init_body,
        grid=(N // _BN,),
        in_specs=[
            pl.BlockSpec((_BN, H), lambda i: (i, 0)),
            pl.BlockSpec((H, 384), lambda i: (0, 0)),
            pl.BlockSpec((_NC, _BN, H), lambda i: (0, i, 0)),
        ],
        out_specs=[
            pl.BlockSpec((_BN, 256), lambda i: (i, 0)),
            pl.BlockSpec((_BN, H), lambda i: (i, 0)),
            pl.BlockSpec((_BN, H), lambda i: (i, 0)),
        ],
        out_shape=[
            jax.ShapeDtypeStruct((N, 256), _f32),
            jax.ShapeDtypeStruct((N, H), _f32),
            jax.ShapeDtypeStruct((N, H), _f32),
        ],
    )(x, mcat, degp)


def _edge_a_body(grn_ref, gc_ref, he_ref, m1_ref, m2_ref, v1_ref, v2_ref,
                 u1_ref, h2_ref):
    grn = grn_ref[...]
    u1 = jnp.maximum(
        grn[:, :H] + gc_ref[...] + _mm(he_ref[...], m1_ref[...]) + v1_ref[...], 0.0)
    u1_ref[...] = u1
    h2_ref[...] = jnp.maximum(
        grn[:, H:] + _mm(u1, m2_ref[...]) + v2_ref[...], 0.0)


def _edge_a(grn, gc, he, m1, m2, v1, v2):
    dh = he.shape[1]
    return pl.pallas_call(
        _edge_a_body,
        grid=(E // _BE,),
        in_specs=[
            pl.BlockSpec((_BE, 256), lambda i: (i, 0)),
            pl.BlockSpec((_BE, H), lambda i: (i, 0)),
            pl.BlockSpec((_BE, dh), lambda i: (i, 0)),
            pl.BlockSpec((dh, H), lambda i: (0, 0)),
            pl.BlockSpec((H, H), lambda i: (0, 0)),
            pl.BlockSpec((1, H), lambda i: (0, 0)),
            pl.BlockSpec((1, H), lambda i: (0, 0)),
        ],
        out_specs=[
            pl.BlockSpec((_BE, H), lambda i: (i, 0)),
            pl.BlockSpec((_BE, H), lambda i: (i, 0)),
        ],
        out_shape=[
            jax.ShapeDtypeStruct((E, H), _f32),
            jax.ShapeDtypeStruct((E, H), _f32),
        ],
    )(grn, gc, he, m1, m2, v1, v2)


def _edge_b_body(gn_ref, u1_ref, m_ref, v_ref, o_ref):
    o_ref[...] = jnp.maximum(
        gn_ref[...] + _mm(u1_ref[...], m_ref[...]) + v_ref[...], 0.0)


def _edge_b(gn, u1, m, v):
    return pl.pallas_call(
        _edge_b_body,
        grid=(E // _BE,),
        in_specs=[
            pl.BlockSpec((_BE, H), lambda i: (i, 0)),
            pl.BlockSpec((_BE, H), lambda i: (i, 0)),
            pl.BlockSpec((H, H), lambda i: (0, 0)),
            pl.BlockSpec((1, H), lambda i: (0, 0)),
        ],
        out_specs=pl.BlockSpec((_BE, H), lambda i: (i, 0)),
        out_shape=jax.ShapeDtypeStruct((E, H), _f32),
    )(gn, u1, m, v)


def _edge_c_body(g1_ref, g2_ref, u1_ref, m_ref, v_ref, o_ref):
    o_ref[...] = jnp.maximum(
        g1_ref[...] + g2_ref[...] + _mm(u1_ref[...], m_ref[...]) + v_ref[...], 0.0)


def _edge_c(g1, g2, u1, m, v):
    return pl.pallas_call(
        _edge_c_body,
        grid=(E // _BE,),
        in_specs=[
            pl.BlockSpec((_BE, H), lambda i: (i, 0)),
            pl.BlockSpec((_BE, H), lambda i: (i, 0)),
            pl.BlockSpec((_BE, H), lambda i: (i, 0)),
            pl.BlockSpec((H, H), lambda i: (0, 0)),
            pl.BlockSpec((1, H), lambda i: (0, 0)),
        ],
        out_specs=pl.BlockSpec((_BE, H), lambda i: (i, 0)),
        out_shape=jax.ShapeDtypeStruct((E, H), _f32),
    )(g1, g2, u1, m, v)


def _node1_body(p_ref, deg_ref, m_ref, v_ref, o_ref):
    s = p_ref[0] + p_ref[1]
    o_ref[...] = _mm(s, m_ref[...]) + deg_ref[:, 0:1] * v_ref[...]


def _node1(p, deg16, m, v):
    return pl.pallas_call(
        _node1_body,
        grid=(N // _BN,),
        in_specs=[
            pl.BlockSpec((_NC, _BN, H), lambda i: (0, i, 0)),
            pl.BlockSpec((_BN, H), lambda i: (i, 0)),
            pl.BlockSpec((H, H), lambda i: (0, 0)),
            pl.BlockSpec((1, H), lambda i: (0, 0)),
        ],
        out_specs=pl.BlockSpec((_BN, H), lambda i: (i, 0)),
        out_shape=jax.ShapeDtypeStruct((N, H), _f32),
    )(p, deg16, m, v)


def _node2_body(p_ref, deg_ref, m_ref, v_ref, trn_ref, tc_ref, tr4_ref, tc4_ref):
    s = p_ref[0] + p_ref[1]
    t = _mm(s, m_ref[...]) + deg_ref[:, 0:1] * v_ref[...]
    trn_ref[...] = t[:, 0:256]
    tc_ref[...] = t[:, 256:384]
    tr4_ref[...] = t[:, 384:512]
    tc4_ref[...] = t[:, 512:640]


def _node2(p, deg16, mcat, vcat):
    return pl.pallas_call(
        _node2_body,
        grid=(N // _BN,),
        in_specs=[
            pl.BlockSpec((_NC, _BN, H), lambda i: (0, i, 0)),
            pl.BlockSpec((_BN, H), lambda i: (i, 0)),
            pl.BlockSpec((H, 640), lambda i: (0, 0)),
            pl.BlockSpec((1, 640), lambda i: (0, 0)),
        ],
        out_specs=[
            pl.BlockSpec((_BN, 256), lambda i: (i, 0)),
            pl.BlockSpec((_BN, H), lambda i: (i, 0)),
            pl.BlockSpec((_BN, H), lambda i: (i, 0)),
            pl.BlockSpec((_BN, H), lambda i: (i, 0)),
        ],
        out_shape=[
            jax.ShapeDtypeStruct((N, 256), _f32),
            jax.ShapeDtypeStruct((N, H), _f32),
            jax.ShapeDtypeStruct((N, H), _f32),
            jax.ShapeDtypeStruct((N, H), _f32),
        ],
    )(p, deg16, mcat, vcat)


def _add2_body(p_ref, o_ref):
    o_ref[...] = p_ref[0] + p_ref[1]


def _add2(p):
    return pl.pallas_call(
        _add2_body,
        grid=(N // _BN,),
        in_specs=[pl.BlockSpec((_NC, _BN, H), lambda i: (0, i, 0))],
        out_specs=pl.BlockSpec((_BN, H), lambda i: (i, 0)),
        out_shape=jax.ShapeDtypeStruct((N, H), _f32),
    )(p)


def _final_body(sp_ref, sdp_ref, b2_ref, bb2_ref, w1t_ref, b1_ref, w2t_ref,
                b2r_ref, o_ref):
    s = sp_ref[0] + sp_ref[1]
    sd = sdp_ref[0][:, 0:1] + sdp_ref[1][:, 0:1]
    g = _mm(s, b2_ref[...]) + sd * bb2_ref[...]
    a = jnp.maximum(_mm(g, w1t_ref[...]) + b1_ref[...], 0.0)
    o_ref[...] = _mm(a, w2t_ref[...]) + b2r_ref[...]


def _final(sp, sdp, b2, bb2, w1t, b1, w2t, b2r):
    return pl.pallas_call(
        _final_body,
        out_shape=jax.ShapeDtypeStruct((G, 1), _f32),
    )(sp, sdp, b2, bb2, w1t, b1, w2t, b2r)


def _hp(a, b):
    return jnp.dot(a, b, precision=lax.Precision.HIGHEST)


def kernel(x, edge_attr, edge_index, batch, edge_enc_W, edge_enc_b,
           etn_edge_W1, etn_edge_b1, etn_edge_W2, etn_edge_b2,
           etn_node_W1, etn_node_b1, etn_node_W2, etn_node_b2,
           nte_node_W1, nte_node_b1, nte_node_W2, nte_node_b2,
           nte_edge_W1, nte_edge_b1, nte_edge_W2, nte_edge_b2,
           reg_W1, reg_b1, reg_W2, reg_b2):
    row = edge_index[0]
    col = edge_index[1]
    col3 = col.reshape(_NW, _NCHS, _CHS)
    batch2 = batch.reshape(125, _CHS)

    sp3e = [jnp.split(etn_edge_W1[l].T, 3, axis=0) for l in range(L)]
    sp2n = [jnp.split(etn_node_W1[l].T, 2, axis=0) for l in range(L)]
    sp2n2 = [jnp.split(nte_node_W1[l].T, 2, axis=0) for l in range(L)]
    sp3e2 = [jnp.split(nte_edge_W1[l].T, 3, axis=0) for l in range(L)]

    degp = _deg(col3)

    mcat0 = jnp.concatenate([sp3e[0][0], sp2n[0][0], sp3e[0][1]], axis=1)
    t_rn, t_c, deg16 = _proj_init(x, mcat0, degp)

    he = edge_attr
    a_fold = edge_enc_W.T
    c_fold = edge_enc_b

    out = None
    for l in range(L):
        rr_e, rc_e, re_e = sp3e[l]
        rx_n, re_n = sp2n[l]
        rx_n2, re_n2 = sp2n2[l]
        w2e_t = etn_edge_W2[l].T

        m1 = _hp(a_fold, re_e)
        v1 = (_hp(c_fold.reshape(1, -1), re_e) + etn_edge_b1[l]).reshape(1, H)
        m2 = _hp(w2e_t, re_n)
        v2 = (_hp(etn_edge_b2[l].reshape(1, -1), re_n) + etn_node_b1[l]).reshape(1, H)
        m3 = _hp(w2e_t, re_n2)
        v3 = (_hp(etn_edge_b2[l].reshape(1, -1), re_n2) + nte_node_b1[l]).reshape(1, H)

        grn = _gather256(t_rn, row)
        gc = _gather128(t_c, col)
        u1, h2 = _edge_a(grn, gc, he, m1, m2, v1, v2)
        s1p = _scatter(h2, col3)

        m5 = _hp(etn_node_W2[l].T, rx_n2)
        v5 = _hp(etn_node_b2[l].reshape(1, -1), rx_n2).reshape(1, H)
        t_n3 = _node1(s1p, deg16, m5, v5)
        gn3 = _gather128(t_n3, row)
        h3 = _edge_b(gn3, u1, m3, v3)
        s2p = _scatter(h3, col3)

        if l < L - 1:
            rr_e2, rc_e2, _ = sp3e2[l]
            b_mat = nte_node_W2[l].T
            bb = nte_node_b2[l].reshape(1, -1)
            nrr_e, nrc_e, _ = sp3e[l + 1]
            nrx_n, _ = sp2n[l + 1]
            mcat = jnp.concatenate(
                [_hp(b_mat, nrr_e), _hp(b_mat, nrx_n), _hp(b_mat, nrc_e),
                 _hp(b_mat, rr_e2), _hp(b_mat, rc_e2)], axis=1)
            vcat = jnp.concatenate(
                [_hp(bb, nrr_e), _hp(bb, nrx_n), _hp(bb, nrc_e),
                 _hp(bb, rr_e2), _hp(bb, rc_e2)], axis=1)
            t_rn, t_c, t_r4, t_c4 = _node2(s2p, deg16, mcat, vcat)

            m4 = _hp(w2e_t, sp3e2[l][2])
            v4 = (_hp(etn_edge_b2[l].reshape(1, -1), sp3e2[l][2])
                  + nte_edge_b1[l]).reshape(1, H)
            gr4 = _gather128(t_r4, row)
            gc4 = _gather128(t_c4, col)
            he = _edge_c(gr4, gc4, u1, m4, v4)
            a_fold = nte_edge_W2[l].T
            c_fold = nte_edge_b2[l]
        else:
            s2 = _add2(s2p)
            sp, sdp = _pool(s2, deg16, batch2)
            out = _final(sp, sdp, nte_node_W2[l].T, nte_node_b2[l].reshape(1, H),
                         reg_W1.T, reg_b1.reshape(1, H), reg_W2.T,
                         reg_b2.reshape(1, 1))
    return out.reshape(G)
